# baseline (speedup 1.0000x reference)
import functools

import jax
import jax.numpy as jnp
from jax import lax
from jax.experimental import pallas as pl
from jax.experimental.pallas import tpu as pltpu
from jax.experimental.pallas import tpu_sc as plsc

_CH = 128
_BLK = 16


@functools.lru_cache(maxsize=None)
def _make_seg_sum(N, D, cpw, NC, NS):
    NPAD = N + 16
    RPS = 8 * (-(-N // (NS * 8)))
    LAST = N - (NS - 1) * RPS
    LASTP = NPAD - (NS - 1) * RPS
    mesh = plsc.VectorSubcoreMesh(core_axis_name="c", subcore_axis_name="s")

    @functools.partial(
        pl.kernel,
        out_type=jax.ShapeDtypeStruct((NC, N, D), jnp.float32),
        mesh=mesh,
        scratch_types=[
            pltpu.VMEM_SHARED((NPAD, D), jnp.float32),
            pltpu.VMEM((2, _CH), jnp.int32),
            pltpu.VMEM((2, _CH), jnp.int32),
            pltpu.VMEM((2, _CH, D), jnp.float32),
            pltpu.SemaphoreType.DMA,
            pltpu.SemaphoreType.DMA,
        ],
    )
    def seg_sum(h_hbm, zeros_hbm, src_hbm, dst_hbm, out_hbm,
                acc, sidx, didx, rows, sem, sem2):
        c = lax.axis_index("c")
        s = lax.axis_index("s")
        wid = s * NC + c

        @pl.when(jnp.logical_and(c == 0, s < NS - 1))
        def _():
            pltpu.sync_copy(h_hbm.at[pl.ds(s * RPS, RPS)],
                            acc.at[pl.ds(s * RPS, RPS)])

        @pl.when(jnp.logical_and(c == 0, s == NS - 1))
        def _():
            pltpu.sync_copy(h_hbm.at[pl.ds((NS - 1) * RPS, LAST)],
                            acc.at[pl.ds((NS - 1) * RPS, LAST)])
            pltpu.sync_copy(zeros_hbm.at[pl.ds(0, NPAD - N)],
                            acc.at[pl.ds(N, NPAD - N)])

        @pl.when(jnp.logical_and(c != 0, s < NS - 1))
        def _():
            pltpu.sync_copy(zeros_hbm.at[pl.ds(s * RPS, RPS)],
                            acc.at[pl.ds(s * RPS, RPS)])

        @pl.when(jnp.logical_and(c != 0, s == NS - 1))
        def _():
            pltpu.sync_copy(zeros_hbm.at[pl.ds((NS - 1) * RPS, LASTP)],
                            acc.at[pl.ds((NS - 1) * RPS, LASTP)])

        base = wid * cpw

        def body(i, carry):
            b = lax.rem(i, 2)
            pltpu.sync_copy(src_hbm.at[base + i], sidx.at[b])
            pltpu.sync_copy(dst_hbm.at[base + i], didx.at[b])
            pltpu.async_copy(h_hbm.at[sidx.at[b]], rows.at[b], sem).wait()
            pltpu.sync_copy(rows.at[b], acc.at[didx.at[b]], add=True)
            return carry

        lax.fori_loop(0, cpw, body, 0)
        plsc.subcore_barrier()

        @pl.when(s < NS - 1)
        def _():
            pltpu.sync_copy(acc.at[pl.ds(s * RPS, RPS)],
                            out_hbm.at[c, pl.ds(s * RPS, RPS)])

        @pl.when(s == NS - 1)
        def _():
            pltpu.sync_copy(acc.at[pl.ds((NS - 1) * RPS, LAST)],
                            out_hbm.at[c, pl.ds((NS - 1) * RPS, LAST)])

    return seg_sum


@functools.lru_cache(maxsize=None)
def _make_gin_matmul(N, D, BN):
    def body(p_ref, w_ref, b_ref, o_ref):
        ss = p_ref[0] + p_ref[1]
        o_ref[...] = jnp.maximum(
            jnp.dot(ss, w_ref[...], preferred_element_type=jnp.float32)
            + b_ref[...], 0.0)

    return pl.pallas_call(
        body,
        grid=(N // BN,),
        in_specs=[
            pl.BlockSpec((2, BN, D), lambda i: (0, i, 0)),
            pl.BlockSpec((D, D), lambda i: (0, 0)),
            pl.BlockSpec((1, D), lambda i: (0, 0)),
        ],
        out_specs=pl.BlockSpec((BN, D), lambda i: (i, 0)),
        out_shape=jax.ShapeDtypeStruct((N, D), jnp.float32),
    )


@functools.lru_cache(maxsize=None)
def _make_final(N, D, O, BN):
    def body(p_ref, w3_ref, b3_ref, x1_ref, x2_ref, wo_ref, bo_ref, o_ref):
        x3 = jnp.maximum(
            jnp.dot(p_ref[0] + p_ref[1], w3_ref[...],
                    preferred_element_type=jnp.float32) + b3_ref[...], 0.0)
        h = jnp.concatenate([x1_ref[...], x2_ref[...], x3], axis=1)
        o_ref[...] = (jnp.dot(h, wo_ref[...],
                              preferred_element_type=jnp.float32)
                      + bo_ref[...])

    return pl.pallas_call(
        body,
        grid=(N // BN,),
        in_specs=[
            pl.BlockSpec((2, BN, D), lambda i: (0, i, 0)),
            pl.BlockSpec((D, D), lambda i: (0, 0)),
            pl.BlockSpec((1, D), lambda i: (0, 0)),
            pl.BlockSpec((BN, D), lambda i: (i, 0)),
            pl.BlockSpec((BN, D), lambda i: (i, 0)),
            pl.BlockSpec((3 * D, O), lambda i: (0, 0)),
            pl.BlockSpec((1, O), lambda i: (0, 0)),
        ],
        out_specs=pl.BlockSpec((BN, O), lambda i: (i, 0)),
        out_shape=jax.ShapeDtypeStruct((N, O), jnp.float32),
    )


def kernel(x, edge_index, W1, b1, W2, b2, W3, b3, Wout, bout):
    N, D = x.shape
    E = edge_index.shape[1]
    O = Wout.shape[1]
    NC, NS = 2, 16
    NW = NC * NS
    cpw = 8 * (-(-E // (NW * _CH * 8)))
    Epad = cpw * NW * _CH
    NPAD = N + 16

    src = edge_index[0]
    dst = edge_index[1]
    pad = Epad - E + 8 * _CH
    src_p = jnp.concatenate(
        [src, jnp.zeros((pad,), jnp.int32)]).reshape(-1, _CH)
    dst_p = jnp.concatenate(
        [dst, N + (jnp.arange(pad, dtype=jnp.int32) % 16)]).reshape(-1, _CH)
    zeros = jnp.zeros((NPAD, D), jnp.float32)

    seg = _make_seg_sum(N, D, cpw, NC, NS)
    mm = _make_gin_matmul(N, D, 1000)
    fin = _make_final(N, D, O, 1000)

    p1 = seg(x, zeros, src_p, dst_p)
    x1 = mm(p1, W1, b1.reshape(1, D))
    p2 = seg(x1, zeros, src_p, dst_p)
    x2 = mm(p2, W2, b2.reshape(1, D))
    p3 = seg(x2, zeros, src_p, dst_p)
    out = fin(p3, W3, b3.reshape(1, D), x1, x2, Wout, bout.reshape(1, O))
    return out


# speedup vs baseline: 1.4963x; 1.4963x over previous
import functools

import jax
import jax.numpy as jnp
from jax import lax
from jax.experimental import pallas as pl
from jax.experimental.pallas import tpu as pltpu
from jax.experimental.pallas import tpu_sc as plsc

_CH = 128
_BLK = 16


@functools.lru_cache(maxsize=None)
def _make_seg_sum(N, D, cpw, NC, NS):
    NPAD = N + 16
    RPS = 8 * (-(-N // (NS * 8)))
    LAST = N - (NS - 1) * RPS
    LASTP = NPAD - (NS - 1) * RPS
    mesh = plsc.VectorSubcoreMesh(core_axis_name="c", subcore_axis_name="s")

    @functools.partial(
        pl.kernel,
        out_type=jax.ShapeDtypeStruct((NC, N, D), jnp.float32),
        mesh=mesh,
        scratch_types=[
            pltpu.VMEM_SHARED((NPAD, D), jnp.float32),
            pltpu.VMEM((2, _CH), jnp.int32),
            pltpu.VMEM((2, _CH), jnp.int32),
            pltpu.VMEM((2, _CH, D), jnp.float32),
            pltpu.SemaphoreType.DMA,
        ],
    )
    def seg_sum(h_hbm, zeros_hbm, src_hbm, dst_hbm, out_hbm,
                acc, sidx, didx, rows, sem):
        c = lax.axis_index("c")
        s = lax.axis_index("s")
        wid = s * NC + c

        @pl.when(jnp.logical_and(c == 0, s < NS - 1))
        def _():
            pltpu.sync_copy(h_hbm.at[pl.ds(s * RPS, RPS)],
                            acc.at[pl.ds(s * RPS, RPS)])

        @pl.when(jnp.logical_and(c == 0, s == NS - 1))
        def _():
            pltpu.sync_copy(h_hbm.at[pl.ds((NS - 1) * RPS, LAST)],
                            acc.at[pl.ds((NS - 1) * RPS, LAST)])
            pltpu.sync_copy(zeros_hbm.at[pl.ds(0, NPAD - N)],
                            acc.at[pl.ds(N, NPAD - N)])

        @pl.when(jnp.logical_and(c != 0, s < NS - 1))
        def _():
            pltpu.sync_copy(zeros_hbm.at[pl.ds(s * RPS, RPS)],
                            acc.at[pl.ds(s * RPS, RPS)])

        @pl.when(jnp.logical_and(c != 0, s == NS - 1))
        def _():
            pltpu.sync_copy(zeros_hbm.at[pl.ds((NS - 1) * RPS, LASTP)],
                            acc.at[pl.ds((NS - 1) * RPS, LASTP)])

        base = wid * cpw

        def body(i, carry):
            b = lax.rem(i, 2)
            pltpu.sync_copy(src_hbm.at[base + i], sidx.at[b])
            pltpu.sync_copy(dst_hbm.at[base + i], didx.at[b])
            pltpu.async_copy(h_hbm.at[sidx.at[b]], rows.at[b], sem).wait()
            pltpu.sync_copy(rows.at[b], acc.at[didx.at[b]], add=True)
            return carry

        lax.fori_loop(0, cpw, body, 0)
        plsc.subcore_barrier()

        @pl.when(s < NS - 1)
        def _():
            pltpu.sync_copy(acc.at[pl.ds(s * RPS, RPS)],
                            out_hbm.at[c, pl.ds(s * RPS, RPS)])

        @pl.when(s == NS - 1)
        def _():
            pltpu.sync_copy(acc.at[pl.ds((NS - 1) * RPS, LAST)],
                            out_hbm.at[c, pl.ds((NS - 1) * RPS, LAST)])

    return seg_sum


@functools.lru_cache(maxsize=None)
def _make_gin_matmul(N, D, BN):
    def body(p_ref, w_ref, b_ref, o_ref):
        ss = p_ref[0] + p_ref[1]
        o_ref[...] = jnp.maximum(
            jnp.dot(ss, w_ref[...], preferred_element_type=jnp.float32)
            + b_ref[...], 0.0)

    return pl.pallas_call(
        body,
        grid=(N // BN,),
        in_specs=[
            pl.BlockSpec((2, BN, D), lambda i: (0, i, 0)),
            pl.BlockSpec((D, D), lambda i: (0, 0)),
            pl.BlockSpec((1, D), lambda i: (0, 0)),
        ],
        out_specs=pl.BlockSpec((BN, D), lambda i: (i, 0)),
        out_shape=jax.ShapeDtypeStruct((N, D), jnp.float32),
    )


@functools.lru_cache(maxsize=None)
def _make_final(N, D, O, BN):
    def body(p_ref, w3_ref, b3_ref, x1_ref, x2_ref, wo_ref, bo_ref, o_ref):
        x3 = jnp.maximum(
            jnp.dot(p_ref[0] + p_ref[1], w3_ref[...],
                    preferred_element_type=jnp.float32) + b3_ref[...], 0.0)
        h = jnp.concatenate([x1_ref[...], x2_ref[...], x3], axis=1)
        o_ref[...] = (jnp.dot(h, wo_ref[...],
                              preferred_element_type=jnp.float32)
                      + bo_ref[...])

    return pl.pallas_call(
        body,
        grid=(N // BN,),
        in_specs=[
            pl.BlockSpec((2, BN, D), lambda i: (0, i, 0)),
            pl.BlockSpec((D, D), lambda i: (0, 0)),
            pl.BlockSpec((1, D), lambda i: (0, 0)),
            pl.BlockSpec((BN, D), lambda i: (i, 0)),
            pl.BlockSpec((BN, D), lambda i: (i, 0)),
            pl.BlockSpec((3 * D, O), lambda i: (0, 0)),
            pl.BlockSpec((1, O), lambda i: (0, 0)),
        ],
        out_specs=pl.BlockSpec((BN, O), lambda i: (i, 0)),
        out_shape=jax.ShapeDtypeStruct((N, O), jnp.float32),
    )


def kernel(x, edge_index, W1, b1, W2, b2, W3, b3, Wout, bout):
    N, D = x.shape
    E = edge_index.shape[1]
    O = Wout.shape[1]
    NC, NS = 2, 16
    NW = NC * NS
    cpw = -(-E // (NW * _CH))
    Epad = cpw * NW * _CH
    NPAD = N + 16

    src = edge_index[0]
    dst = edge_index[1]
    pad = Epad - E
    src_p = jnp.concatenate(
        [src, jnp.zeros((pad,), jnp.int32)]).reshape(-1, _CH)
    dst_p = jnp.concatenate(
        [dst, jnp.full((pad,), N, jnp.int32)]).reshape(-1, _CH)
    zeros = jnp.zeros((NPAD, D), jnp.float32)

    seg = _make_seg_sum(N, D, cpw, NC, NS)
    mm = _make_gin_matmul(N, D, 1000)
    fin = _make_final(N, D, O, 1000)

    p1 = seg(x, zeros, src_p, dst_p)
    x1 = mm(p1, W1, b1.reshape(1, D))
    p2 = seg(x1, zeros, src_p, dst_p)
    x2 = mm(p2, W2, b2.reshape(1, D))
    p3 = seg(x2, zeros, src_p, dst_p)
    out = fin(p3, W3, b3.reshape(1, D), x1, x2, Wout, bout.reshape(1, O))
    return out


# speedup vs baseline: 2.2857x; 1.5276x over previous
import functools

import jax
import jax.numpy as jnp
from jax import lax
from jax.experimental import pallas as pl
from jax.experimental.pallas import tpu as pltpu
from jax.experimental.pallas import tpu_sc as plsc

_CH = 128
_BLK = 16


@functools.lru_cache(maxsize=None)
def _make_seg_sum(N, D, cpw, NC, NS):
    NPAD = N + 16
    RPS = 8 * (-(-N // (NS * 8)))
    LAST = N - (NS - 1) * RPS
    LASTP = NPAD - (NS - 1) * RPS
    mesh = plsc.VectorSubcoreMesh(core_axis_name="c", subcore_axis_name="s")

    @functools.partial(
        pl.kernel,
        out_type=jax.ShapeDtypeStruct((NC, N, D), jnp.float32),
        mesh=mesh,
        scratch_types=[
            pltpu.VMEM_SHARED((NPAD, D), jnp.float32),
            pltpu.VMEM((2, _CH), jnp.int32),
            pltpu.VMEM((2, _CH), jnp.int32),
            pltpu.VMEM((2, _CH, D), jnp.float32),
            pltpu.SemaphoreType.DMA,
        ],
    )
    def seg_sum(h_hbm, zeros_hbm, src_hbm, dst_hbm, out_hbm,
                acc, sidx, didx, rows, sem):
        c = lax.axis_index("c")
        s = lax.axis_index("s")
        wid = s * NC + c

        @pl.when(jnp.logical_and(c == 0, s < NS - 1))
        def _():
            pltpu.sync_copy(h_hbm.at[pl.ds(s * RPS, RPS)],
                            acc.at[pl.ds(s * RPS, RPS)])

        @pl.when(jnp.logical_and(c == 0, s == NS - 1))
        def _():
            pltpu.sync_copy(h_hbm.at[pl.ds((NS - 1) * RPS, LAST)],
                            acc.at[pl.ds((NS - 1) * RPS, LAST)])
            pltpu.sync_copy(zeros_hbm.at[pl.ds(0, NPAD - N)],
                            acc.at[pl.ds(N, NPAD - N)])

        @pl.when(jnp.logical_and(c != 0, s < NS - 1))
        def _():
            pltpu.sync_copy(zeros_hbm.at[pl.ds(s * RPS, RPS)],
                            acc.at[pl.ds(s * RPS, RPS)])

        @pl.when(jnp.logical_and(c != 0, s == NS - 1))
        def _():
            pltpu.sync_copy(zeros_hbm.at[pl.ds((NS - 1) * RPS, LASTP)],
                            acc.at[pl.ds((NS - 1) * RPS, LASTP)])

        base = wid * cpw

        def body(i, carry):
            b = lax.rem(i, 2)
            pltpu.sync_copy(src_hbm.at[base + i], sidx.at[b])
            pltpu.sync_copy(dst_hbm.at[base + i], didx.at[b])
            pltpu.async_copy(h_hbm.at[sidx.at[b]], rows.at[b], sem).wait()
            pltpu.sync_copy(rows.at[b], acc.at[didx.at[b]], add=True)
            return carry

        lax.fori_loop(0, cpw, body, 0)
        plsc.subcore_barrier()

        @pl.when(s < NS - 1)
        def _():
            pltpu.sync_copy(acc.at[pl.ds(s * RPS, RPS)],
                            out_hbm.at[c, pl.ds(s * RPS, RPS)])

        @pl.when(s == NS - 1)
        def _():
            pltpu.sync_copy(acc.at[pl.ds((NS - 1) * RPS, LAST)],
                            out_hbm.at[c, pl.ds((NS - 1) * RPS, LAST)])

    return seg_sum


@functools.lru_cache(maxsize=None)
def _make_gin_matmul(N, D, BN):
    def body(p_ref, w_ref, b_ref, o_ref):
        ss = p_ref[0] + p_ref[1]
        o_ref[...] = jnp.maximum(
            jnp.dot(ss, w_ref[...], preferred_element_type=jnp.float32)
            + b_ref[...], 0.0)

    return pl.pallas_call(
        body,
        grid=(N // BN,),
        in_specs=[
            pl.BlockSpec((2, BN, D), lambda i: (0, i, 0)),
            pl.BlockSpec((D, D), lambda i: (0, 0)),
            pl.BlockSpec((1, D), lambda i: (0, 0)),
        ],
        out_specs=pl.BlockSpec((BN, D), lambda i: (i, 0)),
        out_shape=jax.ShapeDtypeStruct((N, D), jnp.float32),
    )


@functools.lru_cache(maxsize=None)
def _make_final(N, D, O, BN):
    def body(p_ref, w3_ref, b3_ref, x1_ref, x2_ref, wo_ref, bo_ref, o_ref):
        x3 = jnp.maximum(
            jnp.dot(p_ref[0] + p_ref[1], w3_ref[...],
                    preferred_element_type=jnp.float32) + b3_ref[...], 0.0)
        h = jnp.concatenate([x1_ref[...], x2_ref[...], x3], axis=1)
        o_ref[...] = (jnp.dot(h, wo_ref[...],
                              preferred_element_type=jnp.float32)
                      + bo_ref[...])

    return pl.pallas_call(
        body,
        grid=(N // BN,),
        in_specs=[
            pl.BlockSpec((2, BN, D), lambda i: (0, i, 0)),
            pl.BlockSpec((D, D), lambda i: (0, 0)),
            pl.BlockSpec((1, D), lambda i: (0, 0)),
            pl.BlockSpec((BN, D), lambda i: (i, 0)),
            pl.BlockSpec((BN, D), lambda i: (i, 0)),
            pl.BlockSpec((3 * D, O), lambda i: (0, 0)),
            pl.BlockSpec((1, O), lambda i: (0, 0)),
        ],
        out_specs=pl.BlockSpec((BN, O), lambda i: (i, 0)),
        out_shape=jax.ShapeDtypeStruct((N, O), jnp.float32),
    )


def kernel(x, edge_index, W1, b1, W2, b2, W3, b3, Wout, bout):
    N, D = x.shape
    E = edge_index.shape[1]
    O = Wout.shape[1]
    NC, NS = 2, 16
    NW = NC * NS
    cpw = -(-E // (NW * _CH))
    Epad = cpw * NW * _CH
    NPAD = N + 16

    src = edge_index[0]
    dst = edge_index[1]
    pad = Epad - E
    spread = jnp.arange(pad, dtype=jnp.int32)
    src_p = jnp.concatenate(
        [src, (spread * 37) % N]).reshape(-1, _CH)
    dst_p = jnp.concatenate(
        [dst, N + spread % 16]).reshape(-1, _CH)
    zeros = jnp.zeros((NPAD, D), jnp.float32)

    seg = _make_seg_sum(N, D, cpw, NC, NS)
    mm = _make_gin_matmul(N, D, 1000)
    fin = _make_final(N, D, O, 1000)

    p1 = seg(x, zeros, src_p, dst_p)
    x1 = mm(p1, W1, b1.reshape(1, D))
    p2 = seg(x1, zeros, src_p, dst_p)
    x2 = mm(p2, W2, b2.reshape(1, D))
    p3 = seg(x2, zeros, src_p, dst_p)
    out = fin(p3, W3, b3.reshape(1, D), x1, x2, Wout, bout.reshape(1, O))
    return out


# speedup vs baseline: 4.5551x; 1.9929x over previous
import functools

import jax
import jax.numpy as jnp
from jax import lax
from jax.experimental import pallas as pl
from jax.experimental.pallas import tpu as pltpu
from jax.experimental.pallas import tpu_sc as plsc

_CH = 128
_BLK = 16


@functools.lru_cache(maxsize=None)
def _make_seg_sum(N, D, cpw, NC, NS):
    NPAD = N + 16
    RPS = 8 * (-(-N // (NS * 8)))
    LAST = N - (NS - 1) * RPS
    LASTP = NPAD - (NS - 1) * RPS
    mesh = plsc.VectorSubcoreMesh(core_axis_name="c", subcore_axis_name="s")

    nblk = cpw // _BLK
    BLKO = _BLK + 8

    @functools.partial(
        pl.kernel,
        out_type=jax.ShapeDtypeStruct((NC, N, D), jnp.float32),
        mesh=mesh,
        scratch_types=[
            pltpu.VMEM_SHARED((NPAD, D), jnp.float32),
            pltpu.VMEM((2, BLKO, _CH), jnp.int32),
            pltpu.VMEM((2, BLKO, _CH), jnp.int32),
            pltpu.VMEM((2, _CH, D), jnp.float32),
            pltpu.SemaphoreType.DMA,
            pltpu.SemaphoreType.DMA,
            pltpu.SemaphoreType.DMA,
        ],
    )
    def seg_sum(h_hbm, zeros_hbm, src_hbm, dst_hbm, out_hbm,
                acc, sidx, didx, rows, gsem0, gsem1, isem):
        c = lax.axis_index("c")
        s = lax.axis_index("s")
        wid = s * NC + c

        @pl.when(jnp.logical_and(c == 0, s < NS - 1))
        def _():
            pltpu.sync_copy(h_hbm.at[pl.ds(s * RPS, RPS)],
                            acc.at[pl.ds(s * RPS, RPS)])

        @pl.when(jnp.logical_and(c == 0, s == NS - 1))
        def _():
            pltpu.sync_copy(h_hbm.at[pl.ds((NS - 1) * RPS, LAST)],
                            acc.at[pl.ds((NS - 1) * RPS, LAST)])
            pltpu.sync_copy(zeros_hbm.at[pl.ds(0, NPAD - N)],
                            acc.at[pl.ds(N, NPAD - N)])

        @pl.when(jnp.logical_and(c != 0, s < NS - 1))
        def _():
            pltpu.sync_copy(zeros_hbm.at[pl.ds(s * RPS, RPS)],
                            acc.at[pl.ds(s * RPS, RPS)])

        @pl.when(jnp.logical_and(c != 0, s == NS - 1))
        def _():
            pltpu.sync_copy(zeros_hbm.at[pl.ds((NS - 1) * RPS, LASTP)],
                            acc.at[pl.ds((NS - 1) * RPS, LASTP)])

        base = wid * cpw
        pltpu.sync_copy(src_hbm.at[pl.ds(base, BLKO)], sidx.at[0])
        pltpu.sync_copy(dst_hbm.at[pl.ds(base, BLKO)], didx.at[0])
        pltpu.async_copy(h_hbm.at[sidx.at[0, 0]], rows.at[0], gsem0)
        pltpu.async_copy(h_hbm.at[sidx.at[0, 1]], rows.at[1], gsem1)

        plsc.subcore_barrier()

        def outer(k, carry):
            kb = lax.rem(k, 2)
            nkb = 1 - kb

            @pl.when(k > 0)
            def _():
                pltpu.make_async_copy(src_hbm.at[pl.ds(0, BLKO)],
                                      sidx.at[kb], isem).wait()
                pltpu.make_async_copy(dst_hbm.at[pl.ds(0, BLKO)],
                                      didx.at[kb], isem).wait()

            @pl.when(k + 1 < nblk)
            def _():
                off = base + (k + 1) * _BLK
                pltpu.async_copy(src_hbm.at[pl.ds(off, BLKO)],
                                 sidx.at[nkb], isem)
                pltpu.async_copy(dst_hbm.at[pl.ds(off, BLKO)],
                                 didx.at[nkb], isem)

            def inner(p, carry2):
                j = 2 * p
                pltpu.make_async_copy(
                    h_hbm.at[sidx.at[kb, j]], rows.at[0], gsem0).wait()
                pltpu.sync_copy(rows.at[0], acc.at[didx.at[kb, j]], add=True)
                pltpu.async_copy(h_hbm.at[sidx.at[kb, j + 2]], rows.at[0],
                                 gsem0)
                pltpu.make_async_copy(
                    h_hbm.at[sidx.at[kb, j + 1]], rows.at[1], gsem1).wait()
                pltpu.sync_copy(rows.at[1], acc.at[didx.at[kb, j + 1]],
                                add=True)
                pltpu.async_copy(h_hbm.at[sidx.at[kb, j + 3]], rows.at[1],
                                 gsem1)
                return carry2

            lax.fori_loop(0, _BLK // 2, inner, 0)
            return carry

        lax.fori_loop(0, nblk, outer, 0)
        pltpu.make_async_copy(h_hbm.at[sidx.at[0, 0]], rows.at[0],
                              gsem0).wait()
        pltpu.make_async_copy(h_hbm.at[sidx.at[0, 1]], rows.at[1],
                              gsem1).wait()
        plsc.subcore_barrier()

        @pl.when(s < NS - 1)
        def _():
            pltpu.sync_copy(acc.at[pl.ds(s * RPS, RPS)],
                            out_hbm.at[c, pl.ds(s * RPS, RPS)])

        @pl.when(s == NS - 1)
        def _():
            pltpu.sync_copy(acc.at[pl.ds((NS - 1) * RPS, LAST)],
                            out_hbm.at[c, pl.ds((NS - 1) * RPS, LAST)])

    return seg_sum


@functools.lru_cache(maxsize=None)
def _make_gin_matmul(N, D, BN):
    def body(p_ref, w_ref, b_ref, o_ref):
        ss = p_ref[0] + p_ref[1]
        o_ref[...] = jnp.maximum(
            jnp.dot(ss, w_ref[...], preferred_element_type=jnp.float32)
            + b_ref[...], 0.0)

    return pl.pallas_call(
        body,
        grid=(N // BN,),
        in_specs=[
            pl.BlockSpec((2, BN, D), lambda i: (0, i, 0)),
            pl.BlockSpec((D, D), lambda i: (0, 0)),
            pl.BlockSpec((1, D), lambda i: (0, 0)),
        ],
        out_specs=pl.BlockSpec((BN, D), lambda i: (i, 0)),
        out_shape=jax.ShapeDtypeStruct((N, D), jnp.float32),
    )


@functools.lru_cache(maxsize=None)
def _make_final(N, D, O, BN):
    def body(p_ref, w3_ref, b3_ref, x1_ref, x2_ref, wo_ref, bo_ref, o_ref):
        x3 = jnp.maximum(
            jnp.dot(p_ref[0] + p_ref[1], w3_ref[...],
                    preferred_element_type=jnp.float32) + b3_ref[...], 0.0)
        h = jnp.concatenate([x1_ref[...], x2_ref[...], x3], axis=1)
        o_ref[...] = (jnp.dot(h, wo_ref[...],
                              preferred_element_type=jnp.float32)
                      + bo_ref[...])

    return pl.pallas_call(
        body,
        grid=(N // BN,),
        in_specs=[
            pl.BlockSpec((2, BN, D), lambda i: (0, i, 0)),
            pl.BlockSpec((D, D), lambda i: (0, 0)),
            pl.BlockSpec((1, D), lambda i: (0, 0)),
            pl.BlockSpec((BN, D), lambda i: (i, 0)),
            pl.BlockSpec((BN, D), lambda i: (i, 0)),
            pl.BlockSpec((3 * D, O), lambda i: (0, 0)),
            pl.BlockSpec((1, O), lambda i: (0, 0)),
        ],
        out_specs=pl.BlockSpec((BN, O), lambda i: (i, 0)),
        out_shape=jax.ShapeDtypeStruct((N, O), jnp.float32),
    )


def kernel(x, edge_index, W1, b1, W2, b2, W3, b3, Wout, bout):
    N, D = x.shape
    E = edge_index.shape[1]
    O = Wout.shape[1]
    NC, NS = 2, 16
    NW = NC * NS
    cpw = 8 * (-(-E // (NW * _CH * 8)))
    Epad = cpw * NW * _CH
    NPAD = N + 16

    src = edge_index[0]
    dst = edge_index[1]
    pad = Epad - E + 8 * _CH
    spread = jnp.arange(pad, dtype=jnp.int32)
    src_p = jnp.concatenate(
        [src, (spread * 37) % N]).reshape(-1, _CH)
    dst_p = jnp.concatenate(
        [dst, N + spread % 16]).reshape(-1, _CH)
    zeros = jnp.zeros((NPAD, D), jnp.float32)

    seg = _make_seg_sum(N, D, cpw, NC, NS)
    mm = _make_gin_matmul(N, D, 1000)
    fin = _make_final(N, D, O, 1000)

    p1 = seg(x, zeros, src_p, dst_p)
    x1 = mm(p1, W1, b1.reshape(1, D))
    p2 = seg(x1, zeros, src_p, dst_p)
    x2 = mm(p2, W2, b2.reshape(1, D))
    p3 = seg(x2, zeros, src_p, dst_p)
    out = fin(p3, W3, b3.reshape(1, D), x1, x2, Wout, bout.reshape(1, O))
    return out


# speedup vs baseline: 4.6500x; 1.0208x over previous
import functools

import jax
import jax.numpy as jnp
from jax import lax
from jax.experimental import pallas as pl
from jax.experimental.pallas import tpu as pltpu
from jax.experimental.pallas import tpu_sc as plsc

_CH = 128
_BLK = 16


@functools.lru_cache(maxsize=None)
def _make_seg_sum(N, D, cpw, NC, NS):
    NPAD = N + 16
    RPS = 8 * (-(-N // (NS * 8)))
    LAST = N - (NS - 1) * RPS
    LASTP = NPAD - (NS - 1) * RPS
    mesh = plsc.VectorSubcoreMesh(core_axis_name="c", subcore_axis_name="s")

    nblk = cpw // _BLK
    BLKO = _BLK + 8

    @functools.partial(
        pl.kernel,
        out_type=jax.ShapeDtypeStruct((NC, N, D), jnp.float32),
        mesh=mesh,
        scratch_types=[
            pltpu.VMEM_SHARED((NPAD, D), jnp.float32),
            pltpu.VMEM((2, BLKO, _CH), jnp.int32),
            pltpu.VMEM((2, BLKO, _CH), jnp.int32),
            pltpu.VMEM((2, _CH, D), jnp.float32),
            pltpu.SemaphoreType.DMA,
            pltpu.SemaphoreType.DMA,
            pltpu.SemaphoreType.DMA,
            pltpu.SemaphoreType.DMA,
        ],
    )
    def seg_sum(h_hbm, zeros_hbm, src_hbm, dst_hbm, out_hbm,
                acc, sidx, didx, rows, gsem0, gsem1, isem, nsem):
        c = lax.axis_index("c")
        s = lax.axis_index("s")
        wid = s * NC + c

        @pl.when(jnp.logical_and(c == 0, s < NS - 1))
        def _():
            pltpu.async_copy(h_hbm.at[pl.ds(s * RPS, RPS)],
                             acc.at[pl.ds(s * RPS, RPS)], nsem)

        @pl.when(jnp.logical_and(c == 0, s == NS - 1))
        def _():
            pltpu.async_copy(h_hbm.at[pl.ds((NS - 1) * RPS, LAST)],
                             acc.at[pl.ds((NS - 1) * RPS, LAST)], nsem)
            pltpu.async_copy(zeros_hbm.at[pl.ds(0, NPAD - N)],
                             acc.at[pl.ds(N, NPAD - N)], nsem)

        @pl.when(jnp.logical_and(c != 0, s < NS - 1))
        def _():
            pltpu.async_copy(zeros_hbm.at[pl.ds(s * RPS, RPS)],
                             acc.at[pl.ds(s * RPS, RPS)], nsem)

        @pl.when(jnp.logical_and(c != 0, s == NS - 1))
        def _():
            pltpu.async_copy(zeros_hbm.at[pl.ds((NS - 1) * RPS, LASTP)],
                             acc.at[pl.ds((NS - 1) * RPS, LASTP)], nsem)

        base = wid * cpw
        pltpu.sync_copy(src_hbm.at[pl.ds(base, BLKO)], sidx.at[0])
        pltpu.sync_copy(dst_hbm.at[pl.ds(base, BLKO)], didx.at[0])
        pltpu.async_copy(h_hbm.at[sidx.at[0, 0]], rows.at[0], gsem0)
        pltpu.async_copy(h_hbm.at[sidx.at[0, 1]], rows.at[1], gsem1)

        @pl.when(jnp.logical_and(c == 0, s < NS - 1))
        def _():
            pltpu.make_async_copy(h_hbm.at[pl.ds(s * RPS, RPS)],
                                  acc.at[pl.ds(s * RPS, RPS)], nsem).wait()

        @pl.when(jnp.logical_and(c == 0, s == NS - 1))
        def _():
            pltpu.make_async_copy(
                h_hbm.at[pl.ds((NS - 1) * RPS, LAST)],
                acc.at[pl.ds((NS - 1) * RPS, LAST)], nsem).wait()
            pltpu.make_async_copy(zeros_hbm.at[pl.ds(0, NPAD - N)],
                                  acc.at[pl.ds(N, NPAD - N)], nsem).wait()

        @pl.when(jnp.logical_and(c != 0, s < NS - 1))
        def _():
            pltpu.make_async_copy(zeros_hbm.at[pl.ds(s * RPS, RPS)],
                                  acc.at[pl.ds(s * RPS, RPS)], nsem).wait()

        @pl.when(jnp.logical_and(c != 0, s == NS - 1))
        def _():
            pltpu.make_async_copy(
                zeros_hbm.at[pl.ds((NS - 1) * RPS, LASTP)],
                acc.at[pl.ds((NS - 1) * RPS, LASTP)], nsem).wait()

        plsc.subcore_barrier()

        def outer(k, carry):
            kb = lax.rem(k, 2)
            nkb = 1 - kb

            @pl.when(k > 0)
            def _():
                pltpu.make_async_copy(src_hbm.at[pl.ds(0, BLKO)],
                                      sidx.at[kb], isem).wait()
                pltpu.make_async_copy(dst_hbm.at[pl.ds(0, BLKO)],
                                      didx.at[kb], isem).wait()

            @pl.when(k + 1 < nblk)
            def _():
                off = base + (k + 1) * _BLK
                pltpu.async_copy(src_hbm.at[pl.ds(off, BLKO)],
                                 sidx.at[nkb], isem)
                pltpu.async_copy(dst_hbm.at[pl.ds(off, BLKO)],
                                 didx.at[nkb], isem)

            def inner(p, carry2):
                j = 2 * p
                pltpu.make_async_copy(
                    h_hbm.at[sidx.at[kb, j]], rows.at[0], gsem0).wait()
                pltpu.sync_copy(rows.at[0], acc.at[didx.at[kb, j]], add=True)
                pltpu.async_copy(h_hbm.at[sidx.at[kb, j + 2]], rows.at[0],
                                 gsem0)
                pltpu.make_async_copy(
                    h_hbm.at[sidx.at[kb, j + 1]], rows.at[1], gsem1).wait()
                pltpu.sync_copy(rows.at[1], acc.at[didx.at[kb, j + 1]],
                                add=True)
                pltpu.async_copy(h_hbm.at[sidx.at[kb, j + 3]], rows.at[1],
                                 gsem1)
                return carry2

            lax.fori_loop(0, _BLK // 2, inner, 0)
            return carry

        lax.fori_loop(0, nblk, outer, 0)
        pltpu.make_async_copy(h_hbm.at[sidx.at[0, 0]], rows.at[0],
                              gsem0).wait()
        pltpu.make_async_copy(h_hbm.at[sidx.at[0, 1]], rows.at[1],
                              gsem1).wait()
        plsc.subcore_barrier()

        @pl.when(s < NS - 1)
        def _():
            pltpu.sync_copy(acc.at[pl.ds(s * RPS, RPS)],
                            out_hbm.at[c, pl.ds(s * RPS, RPS)])

        @pl.when(s == NS - 1)
        def _():
            pltpu.sync_copy(acc.at[pl.ds((NS - 1) * RPS, LAST)],
                            out_hbm.at[c, pl.ds((NS - 1) * RPS, LAST)])

    return seg_sum


@functools.lru_cache(maxsize=None)
def _make_gin_matmul(N, D, BN):
    def body(p_ref, w_ref, b_ref, o_ref):
        ss = p_ref[0] + p_ref[1]
        o_ref[...] = jnp.maximum(
            jnp.dot(ss, w_ref[...], preferred_element_type=jnp.float32)
            + b_ref[...], 0.0)

    return pl.pallas_call(
        body,
        grid=(N // BN,),
        in_specs=[
            pl.BlockSpec((2, BN, D), lambda i: (0, i, 0)),
            pl.BlockSpec((D, D), lambda i: (0, 0)),
            pl.BlockSpec((1, D), lambda i: (0, 0)),
        ],
        out_specs=pl.BlockSpec((BN, D), lambda i: (i, 0)),
        out_shape=jax.ShapeDtypeStruct((N, D), jnp.float32),
    )


@functools.lru_cache(maxsize=None)
def _make_final(N, D, O, BN):
    def body(p_ref, w3_ref, b3_ref, x1_ref, x2_ref, wo_ref, bo_ref, o_ref):
        x3 = jnp.maximum(
            jnp.dot(p_ref[0] + p_ref[1], w3_ref[...],
                    preferred_element_type=jnp.float32) + b3_ref[...], 0.0)
        h = jnp.concatenate([x1_ref[...], x2_ref[...], x3], axis=1)
        o_ref[...] = (jnp.dot(h, wo_ref[...],
                              preferred_element_type=jnp.float32)
                      + bo_ref[...])

    return pl.pallas_call(
        body,
        grid=(N // BN,),
        in_specs=[
            pl.BlockSpec((2, BN, D), lambda i: (0, i, 0)),
            pl.BlockSpec((D, D), lambda i: (0, 0)),
            pl.BlockSpec((1, D), lambda i: (0, 0)),
            pl.BlockSpec((BN, D), lambda i: (i, 0)),
            pl.BlockSpec((BN, D), lambda i: (i, 0)),
            pl.BlockSpec((3 * D, O), lambda i: (0, 0)),
            pl.BlockSpec((1, O), lambda i: (0, 0)),
        ],
        out_specs=pl.BlockSpec((BN, O), lambda i: (i, 0)),
        out_shape=jax.ShapeDtypeStruct((N, O), jnp.float32),
    )


def kernel(x, edge_index, W1, b1, W2, b2, W3, b3, Wout, bout):
    N, D = x.shape
    E = edge_index.shape[1]
    O = Wout.shape[1]
    NC, NS = 2, 16
    NW = NC * NS
    cpw = 8 * (-(-E // (NW * _CH * 8)))
    Epad = cpw * NW * _CH
    NPAD = N + 16

    src = edge_index[0]
    dst = edge_index[1]
    pad = Epad - E + 8 * _CH
    spread = jnp.arange(pad, dtype=jnp.int32)
    src_p = jnp.concatenate(
        [src, (spread * 37) % N]).reshape(-1, _CH)
    dst_p = jnp.concatenate(
        [dst, N + spread % 16]).reshape(-1, _CH)
    zeros = jnp.zeros((NPAD, D), jnp.float32)

    seg = _make_seg_sum(N, D, cpw, NC, NS)
    mm = _make_gin_matmul(N, D, 1000)
    fin = _make_final(N, D, O, 1000)

    p1 = seg(x, zeros, src_p, dst_p)
    x1 = mm(p1, W1, b1.reshape(1, D))
    p2 = seg(x1, zeros, src_p, dst_p)
    x2 = mm(p2, W2, b2.reshape(1, D))
    p3 = seg(x2, zeros, src_p, dst_p)
    out = fin(p3, W3, b3.reshape(1, D), x1, x2, Wout, bout.reshape(1, O))
    return out


# speedup vs baseline: 4.6540x; 1.0009x over previous
import functools

import jax
import jax.numpy as jnp
from jax import lax
from jax.experimental import pallas as pl
from jax.experimental.pallas import tpu as pltpu
from jax.experimental.pallas import tpu_sc as plsc

_CH = 128
_BLK = 16


@functools.lru_cache(maxsize=None)
def _make_seg_sum(N, D, cpw, NC, NS):
    NPAD = N + 16
    RPS = 8 * (-(-N // (NS * 8)))
    LAST = N - (NS - 1) * RPS
    LASTP = NPAD - (NS - 1) * RPS
    mesh = plsc.VectorSubcoreMesh(core_axis_name="c", subcore_axis_name="s")

    nblk = cpw // _BLK
    BLKO = _BLK + 8

    @functools.partial(
        pl.kernel,
        out_type=jax.ShapeDtypeStruct((NC, N, D), jnp.float32),
        mesh=mesh,
        scratch_types=[
            pltpu.VMEM_SHARED((NPAD, D), jnp.float32),
            pltpu.VMEM((2, BLKO, _CH), jnp.int32),
            pltpu.VMEM((2, BLKO, _CH), jnp.int32),
            pltpu.VMEM((2, _CH, D), jnp.float32),
            pltpu.SemaphoreType.DMA,
            pltpu.SemaphoreType.DMA,
            pltpu.SemaphoreType.DMA,
            pltpu.SemaphoreType.DMA,
        ],
    )
    def seg_sum(h_hbm, zeros_hbm, src_hbm, dst_hbm, out_hbm,
                acc, sidx, didx, rows, gsem0, gsem1, isem, nsem):
        c = lax.axis_index("c")
        s = lax.axis_index("s")
        wid = s * NC + c

        @pl.when(jnp.logical_and(c == 0, s < NS - 1))
        def _():
            pltpu.async_copy(h_hbm.at[pl.ds(s * RPS, RPS)],
                             acc.at[pl.ds(s * RPS, RPS)], nsem)

        @pl.when(jnp.logical_and(c == 0, s == NS - 1))
        def _():
            pltpu.async_copy(h_hbm.at[pl.ds((NS - 1) * RPS, LAST)],
                             acc.at[pl.ds((NS - 1) * RPS, LAST)], nsem)
            pltpu.async_copy(zeros_hbm.at[pl.ds(0, NPAD - N)],
                             acc.at[pl.ds(N, NPAD - N)], nsem)

        @pl.when(jnp.logical_and(c != 0, s < NS - 1))
        def _():
            pltpu.async_copy(zeros_hbm.at[pl.ds(s * RPS, RPS)],
                             acc.at[pl.ds(s * RPS, RPS)], nsem)

        @pl.when(jnp.logical_and(c != 0, s == NS - 1))
        def _():
            pltpu.async_copy(zeros_hbm.at[pl.ds((NS - 1) * RPS, LASTP)],
                             acc.at[pl.ds((NS - 1) * RPS, LASTP)], nsem)

        base = wid * cpw
        pltpu.sync_copy(src_hbm.at[pl.ds(base, BLKO)], sidx.at[0])
        pltpu.sync_copy(dst_hbm.at[pl.ds(base, BLKO)], didx.at[0])
        pltpu.async_copy(h_hbm.at[sidx.at[0, 0]], rows.at[0], gsem0)
        pltpu.async_copy(h_hbm.at[sidx.at[0, 1]], rows.at[1], gsem1)

        @pl.when(jnp.logical_and(c == 0, s < NS - 1))
        def _():
            pltpu.make_async_copy(h_hbm.at[pl.ds(s * RPS, RPS)],
                                  acc.at[pl.ds(s * RPS, RPS)], nsem).wait()

        @pl.when(jnp.logical_and(c == 0, s == NS - 1))
        def _():
            pltpu.make_async_copy(
                h_hbm.at[pl.ds((NS - 1) * RPS, LAST)],
                acc.at[pl.ds((NS - 1) * RPS, LAST)], nsem).wait()
            pltpu.make_async_copy(zeros_hbm.at[pl.ds(0, NPAD - N)],
                                  acc.at[pl.ds(N, NPAD - N)], nsem).wait()

        @pl.when(jnp.logical_and(c != 0, s < NS - 1))
        def _():
            pltpu.make_async_copy(zeros_hbm.at[pl.ds(s * RPS, RPS)],
                                  acc.at[pl.ds(s * RPS, RPS)], nsem).wait()

        @pl.when(jnp.logical_and(c != 0, s == NS - 1))
        def _():
            pltpu.make_async_copy(
                zeros_hbm.at[pl.ds((NS - 1) * RPS, LASTP)],
                acc.at[pl.ds((NS - 1) * RPS, LASTP)], nsem).wait()

        plsc.subcore_barrier()

        def outer(k, carry):
            kb = lax.rem(k, 2)
            nkb = 1 - kb

            @pl.when(k > 0)
            def _():
                pltpu.make_async_copy(src_hbm.at[pl.ds(0, BLKO)],
                                      sidx.at[kb], isem).wait()
                pltpu.make_async_copy(dst_hbm.at[pl.ds(0, BLKO)],
                                      didx.at[kb], isem).wait()

            @pl.when(k + 1 < nblk)
            def _():
                off = base + (k + 1) * _BLK
                pltpu.async_copy(src_hbm.at[pl.ds(off, BLKO)],
                                 sidx.at[nkb], isem)
                pltpu.async_copy(dst_hbm.at[pl.ds(off, BLKO)],
                                 didx.at[nkb], isem)

            def inner(p, carry2):
                j = 2 * p
                pltpu.make_async_copy(
                    h_hbm.at[sidx.at[kb, j]], rows.at[0], gsem0).wait()
                pltpu.sync_copy(rows.at[0], acc.at[didx.at[kb, j]], add=True)
                pltpu.async_copy(h_hbm.at[sidx.at[kb, j + 2]], rows.at[0],
                                 gsem0)
                pltpu.make_async_copy(
                    h_hbm.at[sidx.at[kb, j + 1]], rows.at[1], gsem1).wait()
                pltpu.sync_copy(rows.at[1], acc.at[didx.at[kb, j + 1]],
                                add=True)
                pltpu.async_copy(h_hbm.at[sidx.at[kb, j + 3]], rows.at[1],
                                 gsem1)
                return carry2

            lax.fori_loop(0, _BLK // 2, inner, 0)
            return carry

        lax.fori_loop(0, nblk, outer, 0)
        pltpu.make_async_copy(h_hbm.at[sidx.at[0, 0]], rows.at[0],
                              gsem0).wait()
        pltpu.make_async_copy(h_hbm.at[sidx.at[0, 1]], rows.at[1],
                              gsem1).wait()
        plsc.subcore_barrier()

        @pl.when(s < NS - 1)
        def _():
            pltpu.sync_copy(acc.at[pl.ds(s * RPS, RPS)],
                            out_hbm.at[c, pl.ds(s * RPS, RPS)])

        @pl.when(s == NS - 1)
        def _():
            pltpu.sync_copy(acc.at[pl.ds((NS - 1) * RPS, LAST)],
                            out_hbm.at[c, pl.ds((NS - 1) * RPS, LAST)])

    return seg_sum


@functools.lru_cache(maxsize=None)
def _make_gin_matmul(N, D, BN):
    def body(p_ref, w_ref, b_ref, o_ref):
        ss = p_ref[0] + p_ref[1]
        o_ref[...] = jnp.maximum(
            jnp.dot(ss, w_ref[...], preferred_element_type=jnp.float32)
            + b_ref[...], 0.0)

    return pl.pallas_call(
        body,
        grid=(N // BN,),
        in_specs=[
            pl.BlockSpec((2, BN, D), lambda i: (0, i, 0)),
            pl.BlockSpec((D, D), lambda i: (0, 0)),
            pl.BlockSpec((1, D), lambda i: (0, 0)),
        ],
        out_specs=pl.BlockSpec((BN, D), lambda i: (i, 0)),
        out_shape=jax.ShapeDtypeStruct((N, D), jnp.float32),
    )


@functools.lru_cache(maxsize=None)
def _make_partial_out(N, D, O, BN):
    def body(x1_ref, x2_ref, wo1_ref, wo2_ref, bo_ref, o_ref):
        o_ref[...] = (
            jnp.dot(x1_ref[...], wo1_ref[...],
                    preferred_element_type=jnp.float32)
            + jnp.dot(x2_ref[...], wo2_ref[...],
                      preferred_element_type=jnp.float32)
            + bo_ref[...])

    return pl.pallas_call(
        body,
        grid=(N // BN,),
        in_specs=[
            pl.BlockSpec((BN, D), lambda i: (i, 0)),
            pl.BlockSpec((BN, D), lambda i: (i, 0)),
            pl.BlockSpec((D, O), lambda i: (0, 0)),
            pl.BlockSpec((D, O), lambda i: (0, 0)),
            pl.BlockSpec((1, O), lambda i: (0, 0)),
        ],
        out_specs=pl.BlockSpec((BN, O), lambda i: (i, 0)),
        out_shape=jax.ShapeDtypeStruct((N, O), jnp.float32),
    )


@functools.lru_cache(maxsize=None)
def _make_final(N, D, O, BN):
    def body(p_ref, w3_ref, b3_ref, part_ref, wo3_ref, o_ref):
        x3 = jnp.maximum(
            jnp.dot(p_ref[0] + p_ref[1], w3_ref[...],
                    preferred_element_type=jnp.float32) + b3_ref[...], 0.0)
        o_ref[...] = part_ref[...] + jnp.dot(
            x3, wo3_ref[...], preferred_element_type=jnp.float32)

    return pl.pallas_call(
        body,
        grid=(N // BN,),
        in_specs=[
            pl.BlockSpec((2, BN, D), lambda i: (0, i, 0)),
            pl.BlockSpec((D, D), lambda i: (0, 0)),
            pl.BlockSpec((1, D), lambda i: (0, 0)),
            pl.BlockSpec((BN, O), lambda i: (i, 0)),
            pl.BlockSpec((D, O), lambda i: (0, 0)),
        ],
        out_specs=pl.BlockSpec((BN, O), lambda i: (i, 0)),
        out_shape=jax.ShapeDtypeStruct((N, O), jnp.float32),
    )


def kernel(x, edge_index, W1, b1, W2, b2, W3, b3, Wout, bout):
    N, D = x.shape
    E = edge_index.shape[1]
    O = Wout.shape[1]
    NC, NS = 2, 16
    NW = NC * NS
    cpw = 8 * (-(-E // (NW * _CH * 8)))
    Epad = cpw * NW * _CH
    NPAD = N + 16

    src = edge_index[0]
    dst = edge_index[1]
    pad = Epad - E + 8 * _CH
    spread = jnp.arange(pad, dtype=jnp.int32)
    src_p = jnp.concatenate(
        [src, (spread * 37) % N]).reshape(-1, _CH)
    dst_p = jnp.concatenate(
        [dst, N + spread % 16]).reshape(-1, _CH)
    zeros = jnp.zeros((NPAD, D), jnp.float32)

    seg = _make_seg_sum(N, D, cpw, NC, NS)
    mm = _make_gin_matmul(N, D, 1000)
    po = _make_partial_out(N, D, O, 1000)
    fin = _make_final(N, D, O, 1000)

    p1 = seg(x, zeros, src_p, dst_p)
    x1 = mm(p1, W1, b1.reshape(1, D))
    p2 = seg(x1, zeros, src_p, dst_p)
    x2 = mm(p2, W2, b2.reshape(1, D))
    p3 = seg(x2, zeros, src_p, dst_p)
    part = po(x1, x2, Wout[:D], Wout[D:2 * D], bout.reshape(1, O))
    out = fin(p3, W3, b3.reshape(1, D), part, Wout[2 * D:])
    return out
